# Initial kernel scaffold; baseline (speedup 1.0000x reference)
import jax
import jax.numpy as jnp
from jax.experimental import pallas as pl


def kernel(x, edge_index, W1, b1, W2, b2, Wm1, bm1, Wm2, bm2):
    raise NotImplementedError("write your pallas kernel here")


# speedup vs baseline: 2.1924x; 2.1924x over previous
import jax
import jax.numpy as jnp
from jax.experimental import pallas as pl


def _copy_body(x_ref, o_ref):
    o_ref[...] = x_ref[...]


def kernel(x, edge_index, W1, b1, W2, b2, Wm1, bm1, Wm2, bm2):
    num_nodes = x.shape[0]
    src = edge_index[0]
    dst = edge_index[1]
    deg = jnp.zeros((num_nodes,), jnp.float32).at[dst].add(1.0) + 1.0
    dinv = deg ** -0.5

    def gcn(h_in, W, b):
        hs = (h_in @ W) * dinv[:, None]
        agg = jnp.zeros_like(hs).at[dst].add(hs[src])
        return jax.nn.relu(dinv[:, None] * (agg + hs) + b)

    h = gcn(x, W1, b1)
    h = gcn(h, W2, b2)
    A = h @ Wm1[:128] + bm1
    B = h @ Wm1[128:]
    hidden = jax.nn.relu(A[src] + B[dst])
    flows = hidden @ Wm2 + bm2
    return pl.pallas_call(
        _copy_body,
        grid=(100,),
        in_specs=[pl.BlockSpec((3200, 1), lambda i: (i, 0))],
        out_specs=pl.BlockSpec((3200, 1), lambda i: (i, 0)),
        out_shape=jax.ShapeDtypeStruct(flows.shape, flows.dtype),
    )(flows)


# speedup vs baseline: 5.2243x; 2.3829x over previous
import functools

import jax
import jax.numpy as jnp
from jax import lax
from jax.experimental import pallas as pl
from jax.experimental.pallas import tpu as pltpu
from jax.experimental.pallas import tpu_sc as plsc

N = 10000
E = 320000
D = 128
H = 128

PN = 10240
NW = 32
EPW = 10240
PE = NW * EPW
K = 128
NCH = EPW // K
RPS = PN // 16

_mesh = plsc.VectorSubcoreMesh(core_axis_name="c", subcore_axis_name="s")


def _zero_2d(ref, rows):
    z = jnp.zeros((16,), jnp.float32)

    def body(i, _):
        r = i // 8
        c = (i % 8) * 16
        ref[r, pl.ds(c, 16)] = z
        return 0

    lax.fori_loop(0, rows * 8, body, 0)


def _zero_1d(ref, n):
    z = jnp.zeros((16,), jnp.float32)

    def body(i, _):
        ref[pl.ds(i * 16, 16)] = z
        return 0

    lax.fori_loop(0, n // 16, body, 0)


@functools.partial(
    pl.kernel,
    out_type=jax.ShapeDtypeStruct((2, PN), jnp.float32),
    mesh=_mesh,
    compiler_params=pltpu.CompilerParams(needs_layout_passes=False),
    scratch_types=[
        pltpu.VMEM((K,), jnp.int32),
        pltpu.VMEM((K,), jnp.float32),
        pltpu.VMEM((RPS,), jnp.float32),
        pltpu.VMEM_SHARED((PN,), jnp.float32),
    ],
)
def _sc_deg(dst_hbm, out_hbm, idx_v, ones_v, stripe_v, deg_s):
    cid = lax.axis_index("c")
    sid = lax.axis_index("s")
    wid = cid * 16 + sid

    _zero_1d(stripe_v, RPS)
    pltpu.sync_copy(stripe_v, deg_s.at[pl.ds(sid * RPS, RPS)])
    o = jnp.ones((16,), jnp.float32)
    for i in range(K // 16):
        ones_v[pl.ds(i * 16, 16)] = o
    plsc.subcore_barrier()

    def chunk(c, _):
        base = wid * EPW + c * K
        pltpu.sync_copy(dst_hbm.at[pl.ds(base, K)], idx_v)
        pltpu.sync_copy(ones_v, deg_s.at[idx_v], add=True)
        return 0

    lax.fori_loop(0, NCH, chunk, 0)
    plsc.subcore_barrier()
    pltpu.sync_copy(deg_s.at[pl.ds(sid * RPS, RPS)], stripe_v)
    pltpu.sync_copy(stripe_v, out_hbm.at[cid, pl.ds(sid * RPS, RPS)])


@functools.partial(
    pl.kernel,
    out_type=jax.ShapeDtypeStruct((2, PN, H), jnp.float32),
    mesh=_mesh,
    compiler_params=pltpu.CompilerParams(needs_layout_passes=False),
    scratch_types=[
        pltpu.VMEM((K,), jnp.int32),
        pltpu.VMEM((K,), jnp.int32),
        pltpu.VMEM((K, H), jnp.float32),
        pltpu.VMEM_SHARED((PN, H), jnp.float32),
        pltpu.SemaphoreType.DMA,
    ],
)
def _sc_agg(table_hbm, src_hbm, dst_hbm, out_hbm, sidx_v, didx_v, rows_v,
            agg_s, sem):
    cid = lax.axis_index("c")
    sid = lax.axis_index("s")
    wid = cid * 16 + sid

    _zero_2d(rows_v, K)
    for j in range(RPS // K):
        pltpu.sync_copy(rows_v, agg_s.at[pl.ds(sid * RPS + j * K, K)])
    plsc.subcore_barrier()

    def chunk(c, _):
        base = wid * EPW + c * K
        pltpu.sync_copy(src_hbm.at[pl.ds(base, K)], sidx_v)
        pltpu.sync_copy(dst_hbm.at[pl.ds(base, K)], didx_v)
        pltpu.async_copy(table_hbm.at[sidx_v], rows_v, sem).wait()
        pltpu.sync_copy(rows_v, agg_s.at[didx_v], add=True)
        return 0

    lax.fori_loop(0, NCH, chunk, 0)
    plsc.subcore_barrier()
    for j in range(RPS // K):
        off = sid * RPS + j * K
        pltpu.sync_copy(agg_s.at[pl.ds(off, K)], rows_v)
        pltpu.sync_copy(rows_v, out_hbm.at[cid, pl.ds(off, K)])


@functools.partial(
    pl.kernel,
    out_type=jax.ShapeDtypeStruct((PE,), jnp.float32),
    mesh=_mesh,
    compiler_params=pltpu.CompilerParams(needs_layout_passes=False),
    scratch_types=[
        pltpu.VMEM((K,), jnp.int32),
        pltpu.VMEM((K,), jnp.int32),
        pltpu.VMEM((K, H), jnp.float32),
        pltpu.VMEM((K, H), jnp.float32),
        pltpu.VMEM((H,), jnp.float32),
        pltpu.VMEM((16,), jnp.float32),
        pltpu.VMEM((K,), jnp.float32),
        pltpu.SemaphoreType.DMA,
        pltpu.SemaphoreType.DMA,
    ],
)
def _sc_edge(a_hbm, b_hbm, src_hbm, dst_hbm, w2_hbm, bm2_hbm, out_hbm,
             ridx_v, cidx_v, ra_v, rb_v, w_v, bm2_v, o_v, sem_a, sem_b):
    cid = lax.axis_index("c")
    sid = lax.axis_index("s")
    wid = cid * 16 + sid

    pltpu.sync_copy(w2_hbm, w_v)
    pltpu.sync_copy(bm2_hbm, bm2_v)
    bm2 = bm2_v[...][0]
    lanes = jnp.arange(16, dtype=jnp.int32)

    def chunk(c, _):
        base = wid * EPW + c * K
        pltpu.sync_copy(src_hbm.at[pl.ds(base, K)], ridx_v)
        pltpu.sync_copy(dst_hbm.at[pl.ds(base, K)], cidx_v)
        cpa = pltpu.async_copy(a_hbm.at[ridx_v], ra_v, sem_a)
        cpb = pltpu.async_copy(b_hbm.at[cidx_v], rb_v, sem_b)
        cpa.wait()
        cpb.wait()
        for g in range(K // 16):
            evec = g * 16 + lanes

            def jblock(jb, acc):
                wvec = w_v[pl.ds(jb * 16, 16)]
                for jj in range(16):
                    jv = jnp.full((16,), jb * 16 + jj, dtype=jnp.int32)
                    a = plsc.load_gather(ra_v, [evec, jv])
                    b = plsc.load_gather(rb_v, [evec, jv])
                    s = jnp.maximum(a + b, 0.0)
                    acc = acc + s * wvec[jj]
                return acc

            acc = lax.fori_loop(0, H // 16, jblock, jnp.full((16,), bm2))
            o_v[pl.ds(g * 16, 16)] = acc
        pltpu.sync_copy(o_v, out_hbm.at[pl.ds(base, K)])
        return 0

    lax.fori_loop(0, NCH, chunk, 0)


_ROWS = 1024
_GRID = PN // _ROWS


def _tc1_body(deg0_ref, deg1_ref, x_ref, w1_ref, dinv_ref, hs1_ref):
    deg = deg0_ref[...] + deg1_ref[...] + 1.0
    dinv = lax.rsqrt(deg)
    dinv_ref[...] = dinv
    hs1_ref[...] = jnp.dot(x_ref[...], w1_ref[...],
                           preferred_element_type=jnp.float32) * dinv[:, None]


def _tc2_body(a0_ref, a1_ref, hs_ref, dinv_ref, b_ref, w_ref, out_ref):
    dinv = dinv_ref[...][:, None]
    h = jnp.maximum(dinv * (a0_ref[...] + a1_ref[...] + hs_ref[...])
                    + b_ref[...], 0.0)
    out_ref[...] = jnp.dot(h, w_ref[...],
                           preferred_element_type=jnp.float32) * dinv


def _tc3_body(a0_ref, a1_ref, hs_ref, dinv_ref, b_ref, wm1_ref, bm1_ref,
              a_out_ref, b_out_ref):
    dinv = dinv_ref[...][:, None]
    h = jnp.maximum(dinv * (a0_ref[...] + a1_ref[...] + hs_ref[...])
                    + b_ref[...], 0.0)
    a_out_ref[...] = jnp.dot(h, wm1_ref[0:H, :],
                             preferred_element_type=jnp.float32) + bm1_ref[...]
    b_out_ref[...] = jnp.dot(h, wm1_ref[H:2 * H, :],
                             preferred_element_type=jnp.float32)


def _rows_spec():
    return pl.BlockSpec((_ROWS, H), lambda i: (i, 0))


def _vec_spec():
    return pl.BlockSpec((_ROWS,), lambda i: (i,))


def _full_spec(shape):
    return pl.BlockSpec(shape, lambda i: tuple(0 for _ in shape))


def _tc1(deg0, deg1, x_p, W1):
    return pl.pallas_call(
        _tc1_body,
        grid=(_GRID,),
        in_specs=[_vec_spec(), _vec_spec(), _rows_spec(), _full_spec((D, H))],
        out_specs=[_vec_spec(), _rows_spec()],
        out_shape=[jax.ShapeDtypeStruct((PN,), jnp.float32),
                   jax.ShapeDtypeStruct((PN, H), jnp.float32)],
    )(deg0, deg1, x_p, W1)


def _tc2(a0, a1, hs, dinv, b, W):
    return pl.pallas_call(
        _tc2_body,
        grid=(_GRID,),
        in_specs=[_rows_spec(), _rows_spec(), _rows_spec(), _vec_spec(),
                  _full_spec((H,)), _full_spec((H, H))],
        out_specs=_rows_spec(),
        out_shape=jax.ShapeDtypeStruct((PN, H), jnp.float32),
    )(a0, a1, hs, dinv, b, W)


def _tc3(a0, a1, hs, dinv, b, Wm1, bm1):
    return pl.pallas_call(
        _tc3_body,
        grid=(_GRID,),
        in_specs=[_rows_spec(), _rows_spec(), _rows_spec(), _vec_spec(),
                  _full_spec((H,)), _full_spec((2 * H, H)), _full_spec((H,))],
        out_specs=[_rows_spec(), _rows_spec()],
        out_shape=[jax.ShapeDtypeStruct((PN, H), jnp.float32),
                   jax.ShapeDtypeStruct((PN, H), jnp.float32)],
    )(a0, a1, hs, dinv, b, Wm1, bm1)


def kernel(x, edge_index, W1, b1, W2, b2, Wm1, bm1, Wm2, bm2):
    src = edge_index[0]
    dst = edge_index[1]
    pad_ids = (N + (jnp.arange(PE - E, dtype=jnp.int32) % (PN - N)))
    psrc = jnp.concatenate([src, pad_ids])
    pdst = jnp.concatenate([dst, pad_ids])
    x_p = jnp.pad(x, ((0, PN - N), (0, 0)))

    deg_p = _sc_deg(pdst)
    dinv, hs1 = _tc1(deg_p[0], deg_p[1], x_p, W1)
    agg1 = _sc_agg(hs1, psrc, pdst)
    hs2 = _tc2(agg1[0], agg1[1], hs1, dinv, b1, W2)
    agg2 = _sc_agg(hs2, psrc, pdst)
    A, B = _tc3(agg2[0], agg2[1], hs2, dinv, b2, Wm1, bm1)
    pflows = _sc_edge(A, B, psrc, pdst, Wm2.reshape(H),
                      jnp.pad(bm2, (0, 15)))
    return pflows[:E].reshape(E, 1)


# speedup vs baseline: 5.4482x; 1.0429x over previous
import functools

import jax
import jax.numpy as jnp
from jax import lax
from jax.experimental import pallas as pl
from jax.experimental.pallas import tpu as pltpu
from jax.experimental.pallas import tpu_sc as plsc

N = 10000
E = 320000
D = 128
H = 128

PN = 10240
NW = 32
EPW = 10240
PE = NW * EPW
K = 128
NCH = EPW // K
RPS = PN // 16

_mesh = plsc.VectorSubcoreMesh(core_axis_name="c", subcore_axis_name="s")


def _zero_2d(ref, rows):
    z = jnp.zeros((16,), jnp.float32)

    def body(i, _):
        r = i // 8
        c = (i % 8) * 16
        ref[r, pl.ds(c, 16)] = z
        return 0

    lax.fori_loop(0, rows * 8, body, 0)


def _zero_1d(ref, n):
    z = jnp.zeros((16,), jnp.float32)

    def body(i, _):
        ref[pl.ds(i * 16, 16)] = z
        return 0

    lax.fori_loop(0, n // 16, body, 0)


@functools.partial(
    pl.kernel,
    out_type=jax.ShapeDtypeStruct((2, PN), jnp.float32),
    mesh=_mesh,
    compiler_params=pltpu.CompilerParams(needs_layout_passes=False),
    scratch_types=[
        pltpu.VMEM((K,), jnp.int32),
        pltpu.VMEM((K,), jnp.float32),
        pltpu.VMEM((RPS,), jnp.float32),
        pltpu.VMEM_SHARED((PN,), jnp.float32),
    ],
)
def _sc_deg(dst_hbm, out_hbm, idx_v, ones_v, stripe_v, deg_s):
    cid = lax.axis_index("c")
    sid = lax.axis_index("s")
    wid = cid * 16 + sid

    _zero_1d(stripe_v, RPS)
    pltpu.sync_copy(stripe_v, deg_s.at[pl.ds(sid * RPS, RPS)])
    o = jnp.ones((16,), jnp.float32)
    for i in range(K // 16):
        ones_v[pl.ds(i * 16, 16)] = o
    plsc.subcore_barrier()

    def chunk(c, _):
        base = wid * EPW + c * K
        pltpu.sync_copy(dst_hbm.at[pl.ds(base, K)], idx_v)
        pltpu.sync_copy(ones_v, deg_s.at[idx_v], add=True)
        return 0

    lax.fori_loop(0, NCH, chunk, 0)
    plsc.subcore_barrier()
    pltpu.sync_copy(deg_s.at[pl.ds(sid * RPS, RPS)], stripe_v)
    pltpu.sync_copy(stripe_v, out_hbm.at[cid, pl.ds(sid * RPS, RPS)])


@functools.partial(
    pl.kernel,
    out_type=jax.ShapeDtypeStruct((2, PN, H), jnp.float32),
    mesh=_mesh,
    compiler_params=pltpu.CompilerParams(needs_layout_passes=False),
    scratch_types=[
        pltpu.VMEM((K,), jnp.int32),
        pltpu.VMEM((K,), jnp.int32),
        pltpu.VMEM((K, H), jnp.float32),
        pltpu.VMEM_SHARED((PN, H), jnp.float32),
        pltpu.SemaphoreType.DMA,
    ],
)
def _sc_agg(table_hbm, src_hbm, dst_hbm, out_hbm, sidx_v, didx_v, rows_v,
            agg_s, sem):
    cid = lax.axis_index("c")
    sid = lax.axis_index("s")
    wid = cid * 16 + sid

    _zero_2d(rows_v, K)
    for j in range(RPS // K):
        pltpu.sync_copy(rows_v, agg_s.at[pl.ds(sid * RPS + j * K, K)])
    plsc.subcore_barrier()

    def chunk(c, _):
        base = wid * EPW + c * K
        pltpu.sync_copy(src_hbm.at[pl.ds(base, K)], sidx_v)
        pltpu.sync_copy(dst_hbm.at[pl.ds(base, K)], didx_v)
        pltpu.async_copy(table_hbm.at[sidx_v], rows_v, sem).wait()
        pltpu.sync_copy(rows_v, agg_s.at[didx_v], add=True)
        return 0

    lax.fori_loop(0, NCH, chunk, 0)
    plsc.subcore_barrier()
    for j in range(RPS // K):
        off = sid * RPS + j * K
        pltpu.sync_copy(agg_s.at[pl.ds(off, K)], rows_v)
        pltpu.sync_copy(rows_v, out_hbm.at[cid, pl.ds(off, K)])


@functools.partial(
    pl.kernel,
    out_type=jax.ShapeDtypeStruct((PE,), jnp.float32),
    mesh=_mesh,
    compiler_params=pltpu.CompilerParams(needs_layout_passes=False),
    scratch_types=[
        pltpu.VMEM((K,), jnp.int32),
        pltpu.VMEM((K,), jnp.int32),
        pltpu.VMEM((K, H), jnp.float32),
        pltpu.VMEM((K, H), jnp.float32),
        pltpu.VMEM((H,), jnp.float32),
        pltpu.VMEM((16,), jnp.float32),
        pltpu.VMEM((K,), jnp.float32),
        pltpu.SemaphoreType.DMA,
        pltpu.SemaphoreType.DMA,
    ],
)
def _sc_edge(a_hbm, b_hbm, src_hbm, dst_hbm, w2_hbm, bm2_hbm, out_hbm,
             ridx_v, cidx_v, ra_v, rb_v, w_v, bm2_v, o_v, sem_a, sem_b):
    cid = lax.axis_index("c")
    sid = lax.axis_index("s")
    wid = cid * 16 + sid

    pltpu.sync_copy(w2_hbm, w_v)
    pltpu.sync_copy(bm2_hbm, bm2_v)
    bm2 = bm2_v[...][0]
    lanes = jnp.arange(16, dtype=jnp.int32)

    def chunk(c, _):
        base = wid * EPW + c * K
        pltpu.sync_copy(src_hbm.at[pl.ds(base, K)], ridx_v)
        pltpu.sync_copy(dst_hbm.at[pl.ds(base, K)], cidx_v)
        cpa = pltpu.async_copy(a_hbm.at[ridx_v], ra_v, sem_a)
        cpb = pltpu.async_copy(b_hbm.at[cidx_v], rb_v, sem_b)
        cpa.wait()
        cpb.wait()
        def group(g, _):
            evec = g * 16 + lanes
            accs = [jnp.full((16,), bm2), jnp.zeros((16,), jnp.float32),
                    jnp.zeros((16,), jnp.float32), jnp.zeros((16,), jnp.float32)]
            for jb in range(H // 16):
                wvec = w_v[pl.ds(jb * 16, 16)]
                for jj in range(16):
                    j = jb * 16 + jj
                    jv = jnp.full((16,), j, dtype=jnp.int32)
                    a = plsc.load_gather(ra_v, [evec, jv])
                    b = plsc.load_gather(rb_v, [evec, jv])
                    s = jnp.maximum(a + b, 0.0)
                    accs[j % 4] = accs[j % 4] + s * wvec[jj]
            o_v[pl.ds(g * 16, 16)] = (accs[0] + accs[1]) + (accs[2] + accs[3])
            return 0

        lax.fori_loop(0, K // 16, group, 0)
        pltpu.sync_copy(o_v, out_hbm.at[pl.ds(base, K)])
        return 0

    lax.fori_loop(0, NCH, chunk, 0)


_ROWS = 1024
_GRID = PN // _ROWS


def _tc1_body(deg0_ref, deg1_ref, x_ref, w1_ref, dinv_ref, hs1_ref):
    deg = deg0_ref[...] + deg1_ref[...] + 1.0
    dinv = lax.rsqrt(deg)
    dinv_ref[...] = dinv
    hs1_ref[...] = jnp.dot(x_ref[...], w1_ref[...],
                           preferred_element_type=jnp.float32) * dinv[:, None]


def _tc2_body(a0_ref, a1_ref, hs_ref, dinv_ref, b_ref, w_ref, out_ref):
    dinv = dinv_ref[...][:, None]
    h = jnp.maximum(dinv * (a0_ref[...] + a1_ref[...] + hs_ref[...])
                    + b_ref[...], 0.0)
    out_ref[...] = jnp.dot(h, w_ref[...],
                           preferred_element_type=jnp.float32) * dinv


def _tc3_body(a0_ref, a1_ref, hs_ref, dinv_ref, b_ref, wm1_ref, bm1_ref,
              a_out_ref, b_out_ref):
    dinv = dinv_ref[...][:, None]
    h = jnp.maximum(dinv * (a0_ref[...] + a1_ref[...] + hs_ref[...])
                    + b_ref[...], 0.0)
    a_out_ref[...] = jnp.dot(h, wm1_ref[0:H, :],
                             preferred_element_type=jnp.float32) + bm1_ref[...]
    b_out_ref[...] = jnp.dot(h, wm1_ref[H:2 * H, :],
                             preferred_element_type=jnp.float32)


def _rows_spec():
    return pl.BlockSpec((_ROWS, H), lambda i: (i, 0))


def _vec_spec():
    return pl.BlockSpec((_ROWS,), lambda i: (i,))


def _full_spec(shape):
    return pl.BlockSpec(shape, lambda i: tuple(0 for _ in shape))


def _tc1(deg0, deg1, x_p, W1):
    return pl.pallas_call(
        _tc1_body,
        grid=(_GRID,),
        in_specs=[_vec_spec(), _vec_spec(), _rows_spec(), _full_spec((D, H))],
        out_specs=[_vec_spec(), _rows_spec()],
        out_shape=[jax.ShapeDtypeStruct((PN,), jnp.float32),
                   jax.ShapeDtypeStruct((PN, H), jnp.float32)],
    )(deg0, deg1, x_p, W1)


def _tc2(a0, a1, hs, dinv, b, W):
    return pl.pallas_call(
        _tc2_body,
        grid=(_GRID,),
        in_specs=[_rows_spec(), _rows_spec(), _rows_spec(), _vec_spec(),
                  _full_spec((H,)), _full_spec((H, H))],
        out_specs=_rows_spec(),
        out_shape=jax.ShapeDtypeStruct((PN, H), jnp.float32),
    )(a0, a1, hs, dinv, b, W)


def _tc3(a0, a1, hs, dinv, b, Wm1, bm1):
    return pl.pallas_call(
        _tc3_body,
        grid=(_GRID,),
        in_specs=[_rows_spec(), _rows_spec(), _rows_spec(), _vec_spec(),
                  _full_spec((H,)), _full_spec((2 * H, H)), _full_spec((H,))],
        out_specs=[_rows_spec(), _rows_spec()],
        out_shape=[jax.ShapeDtypeStruct((PN, H), jnp.float32),
                   jax.ShapeDtypeStruct((PN, H), jnp.float32)],
    )(a0, a1, hs, dinv, b, Wm1, bm1)


def kernel(x, edge_index, W1, b1, W2, b2, Wm1, bm1, Wm2, bm2):
    src = edge_index[0]
    dst = edge_index[1]
    pad_ids = (N + (jnp.arange(PE - E, dtype=jnp.int32) % (PN - N)))
    psrc = jnp.concatenate([src, pad_ids])
    pdst = jnp.concatenate([dst, pad_ids])
    x_p = jnp.pad(x, ((0, PN - N), (0, 0)))

    deg_p = _sc_deg(pdst)
    dinv, hs1 = _tc1(deg_p[0], deg_p[1], x_p, W1)
    agg1 = _sc_agg(hs1, psrc, pdst)
    hs2 = _tc2(agg1[0], agg1[1], hs1, dinv, b1, W2)
    agg2 = _sc_agg(hs2, psrc, pdst)
    A, B = _tc3(agg2[0], agg2[1], hs2, dinv, b2, Wm1, bm1)
    pflows = _sc_edge(A, B, psrc, pdst, Wm2.reshape(H),
                      jnp.pad(bm2, (0, 15)))
    return pflows[:E].reshape(E, 1)


# speedup vs baseline: 8.5513x; 1.5696x over previous
import functools

import jax
import jax.numpy as jnp
from jax import lax
from jax.experimental import pallas as pl
from jax.experimental.pallas import tpu as pltpu
from jax.experimental.pallas import tpu_sc as plsc

N = 10000
E = 320000
D = 128
H = 128

PN = 10240
NW = 32
EPW = 10240
PE = NW * EPW
K = 128
NCH = EPW // K
RPS = PN // 16

_mesh = plsc.VectorSubcoreMesh(core_axis_name="c", subcore_axis_name="s")


def _zero_2d(ref, rows):
    z = jnp.zeros((16,), jnp.float32)

    def body(i, _):
        r = i // 8
        c = (i % 8) * 16
        ref[r, pl.ds(c, 16)] = z
        return 0

    lax.fori_loop(0, rows * 8, body, 0)


def _zero_1d(ref, n):
    z = jnp.zeros((16,), jnp.float32)

    def body(i, _):
        ref[pl.ds(i * 16, 16)] = z
        return 0

    lax.fori_loop(0, n // 16, body, 0)


@functools.partial(
    pl.kernel,
    out_type=jax.ShapeDtypeStruct((2, PN), jnp.float32),
    mesh=_mesh,
    compiler_params=pltpu.CompilerParams(needs_layout_passes=False),
    scratch_types=[
        pltpu.VMEM((K,), jnp.int32),
        pltpu.VMEM((K,), jnp.float32),
        pltpu.VMEM((RPS,), jnp.float32),
        pltpu.VMEM_SHARED((PN,), jnp.float32),
    ],
)
def _sc_deg(dst_hbm, out_hbm, idx_v, ones_v, stripe_v, deg_s):
    cid = lax.axis_index("c")
    sid = lax.axis_index("s")
    wid = cid * 16 + sid

    _zero_1d(stripe_v, RPS)
    pltpu.sync_copy(stripe_v, deg_s.at[pl.ds(sid * RPS, RPS)])
    o = jnp.ones((16,), jnp.float32)
    for i in range(K // 16):
        ones_v[pl.ds(i * 16, 16)] = o
    plsc.subcore_barrier()

    def chunk(c, _):
        base = wid * EPW + c * K
        pltpu.sync_copy(dst_hbm.at[pl.ds(base, K)], idx_v)
        pltpu.sync_copy(ones_v, deg_s.at[idx_v], add=True)
        return 0

    lax.fori_loop(0, NCH, chunk, 0)
    plsc.subcore_barrier()
    pltpu.sync_copy(deg_s.at[pl.ds(sid * RPS, RPS)], stripe_v)
    pltpu.sync_copy(stripe_v, out_hbm.at[cid, pl.ds(sid * RPS, RPS)])


@functools.partial(
    pl.kernel,
    out_type=jax.ShapeDtypeStruct((2, PN, H), jnp.float32),
    mesh=_mesh,
    compiler_params=pltpu.CompilerParams(needs_layout_passes=False),
    scratch_types=[
        pltpu.VMEM((K,), jnp.int32),
        pltpu.VMEM((K,), jnp.int32),
        pltpu.VMEM((K, H), jnp.float32),
        pltpu.VMEM_SHARED((PN, H), jnp.float32),
        pltpu.SemaphoreType.DMA,
    ],
)
def _sc_agg(table_hbm, src_hbm, dst_hbm, out_hbm, sidx_v, didx_v, rows_v,
            agg_s, sem):
    cid = lax.axis_index("c")
    sid = lax.axis_index("s")
    wid = cid * 16 + sid

    _zero_2d(rows_v, K)
    for j in range(RPS // K):
        pltpu.sync_copy(rows_v, agg_s.at[pl.ds(sid * RPS + j * K, K)])
    plsc.subcore_barrier()

    def chunk(c, _):
        base = wid * EPW + c * K
        pltpu.sync_copy(src_hbm.at[pl.ds(base, K)], sidx_v)
        pltpu.sync_copy(dst_hbm.at[pl.ds(base, K)], didx_v)
        pltpu.async_copy(table_hbm.at[sidx_v], rows_v, sem).wait()
        pltpu.sync_copy(rows_v, agg_s.at[didx_v], add=True)
        return 0

    lax.fori_loop(0, NCH, chunk, 0)
    plsc.subcore_barrier()
    for j in range(RPS // K):
        off = sid * RPS + j * K
        pltpu.sync_copy(agg_s.at[pl.ds(off, K)], rows_v)
        pltpu.sync_copy(rows_v, out_hbm.at[cid, pl.ds(off, K)])


@functools.partial(
    pl.kernel,
    out_type=jax.ShapeDtypeStruct((PE, 16), jnp.float32),
    mesh=_mesh,
    compiler_params=pltpu.CompilerParams(needs_layout_passes=False),
    scratch_types=[
        pltpu.VMEM((K,), jnp.int32),
        pltpu.VMEM((K,), jnp.int32),
        pltpu.VMEM((K, H), jnp.float32),
        pltpu.VMEM((K, H), jnp.float32),
        pltpu.VMEM((H,), jnp.float32),
        pltpu.VMEM((K, 16), jnp.float32),
        pltpu.SemaphoreType.DMA,
        pltpu.SemaphoreType.DMA,
    ],
)
def _sc_edge(a_hbm, b_hbm, src_hbm, dst_hbm, w2_hbm, out_hbm,
             ridx_v, cidx_v, ra_v, rb_v, w_v, o_v, sem_a, sem_b):
    cid = lax.axis_index("c")
    sid = lax.axis_index("s")
    wid = cid * 16 + sid

    pltpu.sync_copy(w2_hbm, w_v)
    wvs = [w_v[pl.ds(jb * 16, 16)] for jb in range(H // 16)]

    def chunk(c, _):
        base = wid * EPW + c * K
        pltpu.sync_copy(src_hbm.at[pl.ds(base, K)], ridx_v)
        pltpu.sync_copy(dst_hbm.at[pl.ds(base, K)], cidx_v)
        cpa = pltpu.async_copy(a_hbm.at[ridx_v], ra_v, sem_a)
        cpb = pltpu.async_copy(b_hbm.at[cidx_v], rb_v, sem_b)
        cpa.wait()
        cpb.wait()

        def pair(p, _):
            for u in range(2):
                e = p * 2 + u
                accs = [jnp.zeros((16,), jnp.float32) for _ in range(4)]
                for jb in range(H // 16):
                    a = ra_v[e, pl.ds(jb * 16, 16)]
                    b = rb_v[e, pl.ds(jb * 16, 16)]
                    s = jnp.maximum(a + b, 0.0)
                    accs[jb % 4] = accs[jb % 4] + s * wvs[jb]
                o_v[e, :] = (accs[0] + accs[1]) + (accs[2] + accs[3])
            return 0

        lax.fori_loop(0, K // 2, pair, 0)
        pltpu.sync_copy(o_v, out_hbm.at[pl.ds(base, K)])
        return 0

    lax.fori_loop(0, NCH, chunk, 0)


def _tc4_body(p_ref, bm2_ref, out_ref):
    out_ref[...] = jnp.sum(p_ref[...], axis=1, keepdims=True) + bm2_ref[0]


_EROWS = 4096


def _tc4(partials, bm2):
    return pl.pallas_call(
        _tc4_body,
        grid=(PE // _EROWS,),
        in_specs=[pl.BlockSpec((_EROWS, 16), lambda i: (i, 0)),
                  pl.BlockSpec((1,), lambda i: (0,))],
        out_specs=pl.BlockSpec((_EROWS, 1), lambda i: (i, 0)),
        out_shape=jax.ShapeDtypeStruct((PE, 1), jnp.float32),
    )(partials, bm2)


_ROWS = 1024
_GRID = PN // _ROWS


def _tc1_body(deg0_ref, deg1_ref, x_ref, w1_ref, dinv_ref, hs1_ref):
    deg = deg0_ref[...] + deg1_ref[...] + 1.0
    dinv = lax.rsqrt(deg)
    dinv_ref[...] = dinv
    hs1_ref[...] = jnp.dot(x_ref[...], w1_ref[...],
                           preferred_element_type=jnp.float32) * dinv[:, None]


def _tc2_body(a0_ref, a1_ref, hs_ref, dinv_ref, b_ref, w_ref, out_ref):
    dinv = dinv_ref[...][:, None]
    h = jnp.maximum(dinv * (a0_ref[...] + a1_ref[...] + hs_ref[...])
                    + b_ref[...], 0.0)
    out_ref[...] = jnp.dot(h, w_ref[...],
                           preferred_element_type=jnp.float32) * dinv


def _tc3_body(a0_ref, a1_ref, hs_ref, dinv_ref, b_ref, wm1_ref, bm1_ref,
              a_out_ref, b_out_ref):
    dinv = dinv_ref[...][:, None]
    h = jnp.maximum(dinv * (a0_ref[...] + a1_ref[...] + hs_ref[...])
                    + b_ref[...], 0.0)
    a_out_ref[...] = jnp.dot(h, wm1_ref[0:H, :],
                             preferred_element_type=jnp.float32) + bm1_ref[...]
    b_out_ref[...] = jnp.dot(h, wm1_ref[H:2 * H, :],
                             preferred_element_type=jnp.float32)


def _rows_spec():
    return pl.BlockSpec((_ROWS, H), lambda i: (i, 0))


def _vec_spec():
    return pl.BlockSpec((_ROWS,), lambda i: (i,))


def _full_spec(shape):
    return pl.BlockSpec(shape, lambda i: tuple(0 for _ in shape))


def _tc1(deg0, deg1, x_p, W1):
    return pl.pallas_call(
        _tc1_body,
        grid=(_GRID,),
        in_specs=[_vec_spec(), _vec_spec(), _rows_spec(), _full_spec((D, H))],
        out_specs=[_vec_spec(), _rows_spec()],
        out_shape=[jax.ShapeDtypeStruct((PN,), jnp.float32),
                   jax.ShapeDtypeStruct((PN, H), jnp.float32)],
    )(deg0, deg1, x_p, W1)


def _tc2(a0, a1, hs, dinv, b, W):
    return pl.pallas_call(
        _tc2_body,
        grid=(_GRID,),
        in_specs=[_rows_spec(), _rows_spec(), _rows_spec(), _vec_spec(),
                  _full_spec((H,)), _full_spec((H, H))],
        out_specs=_rows_spec(),
        out_shape=jax.ShapeDtypeStruct((PN, H), jnp.float32),
    )(a0, a1, hs, dinv, b, W)


def _tc3(a0, a1, hs, dinv, b, Wm1, bm1):
    return pl.pallas_call(
        _tc3_body,
        grid=(_GRID,),
        in_specs=[_rows_spec(), _rows_spec(), _rows_spec(), _vec_spec(),
                  _full_spec((H,)), _full_spec((2 * H, H)), _full_spec((H,))],
        out_specs=[_rows_spec(), _rows_spec()],
        out_shape=[jax.ShapeDtypeStruct((PN, H), jnp.float32),
                   jax.ShapeDtypeStruct((PN, H), jnp.float32)],
    )(a0, a1, hs, dinv, b, Wm1, bm1)


def kernel(x, edge_index, W1, b1, W2, b2, Wm1, bm1, Wm2, bm2):
    src = edge_index[0]
    dst = edge_index[1]
    pad_ids = (N + (jnp.arange(PE - E, dtype=jnp.int32) % (PN - N)))
    psrc = jnp.concatenate([src, pad_ids])
    pdst = jnp.concatenate([dst, pad_ids])
    x_p = jnp.pad(x, ((0, PN - N), (0, 0)))

    deg_p = _sc_deg(pdst)
    dinv, hs1 = _tc1(deg_p[0], deg_p[1], x_p, W1)
    agg1 = _sc_agg(hs1, psrc, pdst)
    hs2 = _tc2(agg1[0], agg1[1], hs1, dinv, b1, W2)
    agg2 = _sc_agg(hs2, psrc, pdst)
    A, B = _tc3(agg2[0], agg2[1], hs2, dinv, b2, Wm1, bm1)
    partials = _sc_edge(A, B, psrc, pdst, Wm2.reshape(H))
    pflows = _tc4(partials, bm2)
    return pflows[:E]


# speedup vs baseline: 12.6041x; 1.4739x over previous
import functools

import jax
import jax.numpy as jnp
from jax import lax
from jax.experimental import pallas as pl
from jax.experimental.pallas import tpu as pltpu
from jax.experimental.pallas import tpu_sc as plsc

N = 10000
E = 320000
D = 128
H = 128

PN = 10240
NW = 32
EPW = 10240
PE = NW * EPW
K = 128
NCH = EPW // K
RPS = PN // 16
FH = H // 2
EPT = PE // 16
NCH2 = EPT // K

_mesh = plsc.VectorSubcoreMesh(core_axis_name="c", subcore_axis_name="s")


def _zero_2d(ref, rows, cols):
    z = jnp.zeros((16,), jnp.float32)
    g = cols // 16

    def body(i, _):
        r = i // g
        c = (i % g) * 16
        ref[r, pl.ds(c, 16)] = z
        return 0

    lax.fori_loop(0, rows * g, body, 0)


def _zero_1d(ref, n):
    z = jnp.zeros((16,), jnp.float32)

    def body(i, _):
        ref[pl.ds(i * 16, 16)] = z
        return 0

    lax.fori_loop(0, n // 16, body, 0)


@functools.partial(
    pl.kernel,
    out_type=jax.ShapeDtypeStruct((2, PN), jnp.float32),
    mesh=_mesh,
    compiler_params=pltpu.CompilerParams(needs_layout_passes=False),
    scratch_types=[
        pltpu.VMEM((K,), jnp.int32),
        pltpu.VMEM((K,), jnp.float32),
        pltpu.VMEM((RPS,), jnp.float32),
        pltpu.VMEM_SHARED((PN,), jnp.float32),
    ],
)
def _sc_deg(dst_hbm, out_hbm, idx_v, ones_v, stripe_v, deg_s):
    cid = lax.axis_index("c")
    sid = lax.axis_index("s")
    wid = cid * 16 + sid

    _zero_1d(stripe_v, RPS)
    pltpu.sync_copy(stripe_v, deg_s.at[pl.ds(sid * RPS, RPS)])
    o = jnp.ones((16,), jnp.float32)
    for i in range(K // 16):
        ones_v[pl.ds(i * 16, 16)] = o
    plsc.subcore_barrier()

    def chunk(c, _):
        base = wid * EPW + c * K
        pltpu.sync_copy(dst_hbm.at[pl.ds(base, K)], idx_v)
        pltpu.sync_copy(ones_v, deg_s.at[idx_v], add=True)
        return 0

    lax.fori_loop(0, NCH, chunk, 0)
    plsc.subcore_barrier()
    pltpu.sync_copy(deg_s.at[pl.ds(sid * RPS, RPS)], stripe_v)
    pltpu.sync_copy(stripe_v, out_hbm.at[cid, pl.ds(sid * RPS, RPS)])


@functools.partial(
    pl.kernel,
    out_type=jax.ShapeDtypeStruct((2, PN, FH), jnp.float32),
    mesh=_mesh,
    compiler_params=pltpu.CompilerParams(needs_layout_passes=False,
                                         use_tc_tiling_on_sc=False),
    scratch_types=[
        pltpu.VMEM((NCH2, K), jnp.int32),
        pltpu.VMEM((NCH2, K), jnp.int32),
        pltpu.VMEM((4, K, FH), jnp.float32),
        pltpu.VMEM_SHARED((PN, FH), jnp.float32),
        pltpu.SemaphoreType.DMA,
        pltpu.SemaphoreType.DMA,
        pltpu.SemaphoreType.DMA,
        pltpu.SemaphoreType.DMA,
    ],
)
def _sc_agg(table_hbm, src_hbm, dst_hbm, out_hbm, sidx_v, didx_v, rows_v,
            agg_s, g0, g1, g2, g3):
    cid = lax.axis_index("c")
    sid = lax.axis_index("s")
    gsem = [g0, g1, g2, g3]
    table_c = table_hbm.at[cid]

    _zero_2d(rows_v.at[0], K, FH)
    for j in range(RPS // K):
        pltpu.sync_copy(rows_v.at[0], agg_s.at[pl.ds(sid * RPS + j * K, K)])
    pltpu.sync_copy(src_hbm.at[sid], sidx_v)
    pltpu.sync_copy(dst_hbm.at[sid], didx_v)
    plsc.subcore_barrier()

    def gather(c, b):
        pltpu.async_copy(table_c.at[sidx_v.at[c]], rows_v.at[b], gsem[b])

    def gwait(b):
        pltpu.make_async_copy(table_c.at[pl.ds(0, K)], rows_v.at[b],
                              gsem[b]).wait()

    def wave(w, sp, prefetch):
        b0, b1 = sp * 2, sp * 2 + 1
        o0, o1 = 2 - sp * 2, 3 - sp * 2
        gwait(b0)
        gwait(b1)
        if prefetch:
            gather(2 * w + 2, o0)
            gather(2 * w + 3, o1)
        pltpu.sync_copy(rows_v.at[b0], agg_s.at[didx_v.at[2 * w]], add=True)
        pltpu.sync_copy(rows_v.at[b1], agg_s.at[didx_v.at[2 * w + 1]],
                        add=True)

    gather(0, 0)
    gather(1, 1)
    wave(0, 0, True)

    def pair(p, _):
        wave(2 * p + 1, 1, True)
        wave(2 * p + 2, 0, True)
        return 0

    lax.fori_loop(0, (NCH2 // 2 - 2) // 2, pair, 0)
    wave(NCH2 // 2 - 1, 1, False)

    plsc.subcore_barrier()
    for j in range(RPS // K):
        off = sid * RPS + j * K
        pltpu.sync_copy(agg_s.at[pl.ds(off, K)], rows_v.at[0])
        pltpu.sync_copy(rows_v.at[0], out_hbm.at[cid, pl.ds(off, K)])


@functools.partial(
    pl.kernel,
    out_type=jax.ShapeDtypeStruct((PE, 16), jnp.float32),
    mesh=_mesh,
    compiler_params=pltpu.CompilerParams(needs_layout_passes=False),
    scratch_types=[
        pltpu.VMEM((NCH, K), jnp.int32),
        pltpu.VMEM((NCH, K), jnp.int32),
        pltpu.VMEM((2, K, H), jnp.float32),
        pltpu.VMEM((2, K, H), jnp.float32),
        pltpu.VMEM((H,), jnp.float32),
        pltpu.VMEM((2, K, 16), jnp.float32),
        pltpu.SemaphoreType.DMA,
        pltpu.SemaphoreType.DMA,
        pltpu.SemaphoreType.DMA,
        pltpu.SemaphoreType.DMA,
        pltpu.SemaphoreType.DMA,
        pltpu.SemaphoreType.DMA,
    ],
)
def _sc_edge(a_hbm, b_hbm, src_hbm, dst_hbm, w2_hbm, out_hbm,
             ridx_v, cidx_v, ra_v, rb_v, w_v, o_v,
             ga0, ga1, gb0, gb1, os0, os1):
    cid = lax.axis_index("c")
    sid = lax.axis_index("s")
    wid = cid * 16 + sid
    gsa = [ga0, ga1]
    gsb = [gb0, gb1]
    osem = [os0, os1]

    pltpu.sync_copy(w2_hbm, w_v)
    pltpu.sync_copy(src_hbm.at[wid], ridx_v)
    pltpu.sync_copy(dst_hbm.at[wid], cidx_v)
    wvs = [w_v[pl.ds(jb * 16, 16)] for jb in range(H // 16)]

    def gather(c, s):
        pltpu.async_copy(a_hbm.at[ridx_v.at[c]], ra_v.at[s], gsa[s])
        pltpu.async_copy(b_hbm.at[cidx_v.at[c]], rb_v.at[s], gsb[s])

    def gwait(s):
        pltpu.make_async_copy(a_hbm.at[pl.ds(0, K)], ra_v.at[s], gsa[s]).wait()
        pltpu.make_async_copy(b_hbm.at[pl.ds(0, K)], rb_v.at[s], gsb[s]).wait()

    def owait(s):
        pltpu.make_async_copy(o_v.at[s], out_hbm.at[pl.ds(0, K)],
                              osem[s]).wait()

    def compute(s):
        ra = ra_v.at[s]
        rb = rb_v.at[s]
        ov = o_v.at[s]

        def pair(p, _):
            for u in range(2):
                e = p * 2 + u
                accs = [jnp.zeros((16,), jnp.float32) for _ in range(4)]
                for jb in range(H // 16):
                    a = ra[e, pl.ds(jb * 16, 16)]
                    b = rb[e, pl.ds(jb * 16, 16)]
                    t = jnp.maximum(a + b, 0.0)
                    accs[jb % 4] = accs[jb % 4] + t * wvs[jb]
                ov[e, :] = (accs[0] + accs[1]) + (accs[2] + accs[3])
            return 0

        lax.fori_loop(0, K // 2, pair, 0)

    def chunk(c, sp, prefetch, wait_out):
        gwait(sp)
        if prefetch:
            gather(c + 1, 1 - sp)
        if wait_out:
            owait(sp)
        compute(sp)
        pltpu.async_copy(o_v.at[sp], out_hbm.at[pl.ds(wid * EPW + c * K, K)],
                         osem[sp])

    gather(0, 0)
    chunk(0, 0, True, False)
    chunk(1, 1, True, False)

    def pair_of_chunks(p, _):
        chunk(2 * p + 2, 0, True, True)
        chunk(2 * p + 3, 1, True, True)
        return 0

    lax.fori_loop(0, (NCH - 4) // 2, pair_of_chunks, 0)
    chunk(NCH - 2, 0, True, True)
    chunk(NCH - 1, 1, False, True)
    owait(0)
    owait(1)


def _tc4_body(p_ref, bm2_ref, out_ref):
    out_ref[...] = jnp.sum(p_ref[...], axis=1, keepdims=True) + bm2_ref[0]


_EROWS = 4096


def _tc4(partials, bm2):
    return pl.pallas_call(
        _tc4_body,
        grid=(PE // _EROWS,),
        in_specs=[pl.BlockSpec((_EROWS, 16), lambda i: (i, 0)),
                  pl.BlockSpec((1,), lambda i: (0,))],
        out_specs=pl.BlockSpec((_EROWS, 1), lambda i: (i, 0)),
        out_shape=jax.ShapeDtypeStruct((PE, 1), jnp.float32),
    )(partials, bm2)


_ROWS = 1024
_GRID = PN // _ROWS


def _tc1_body(deg0_ref, deg1_ref, x_ref, w1_ref, dinv_ref, hs1_ref):
    deg = deg0_ref[...] + deg1_ref[...] + 1.0
    dinv = lax.rsqrt(deg)
    dinv_ref[...] = dinv
    h = jnp.dot(x_ref[...], w1_ref[...],
                preferred_element_type=jnp.float32) * dinv[:, None]
    hs1_ref[0] = h[:, 0:FH]
    hs1_ref[1] = h[:, FH:H]


def _tc2_body(a_ref, hs_ref, dinv_ref, b_ref, w_ref, out_ref):
    dinv = dinv_ref[...][:, None]
    h0 = jnp.maximum(dinv * (a_ref[0] + hs_ref[0]) + b_ref[0:FH], 0.0)
    h1 = jnp.maximum(dinv * (a_ref[1] + hs_ref[1]) + b_ref[FH:H], 0.0)
    h = jnp.concatenate([h0, h1], axis=1)
    t = jnp.dot(h, w_ref[...], preferred_element_type=jnp.float32) * dinv
    out_ref[0] = t[:, 0:FH]
    out_ref[1] = t[:, FH:H]


def _tc3_body(a_ref, hs_ref, dinv_ref, b_ref, wm1_ref, bm1_ref,
              a_out_ref, b_out_ref):
    dinv = dinv_ref[...][:, None]
    h0 = jnp.maximum(dinv * (a_ref[0] + hs_ref[0]) + b_ref[0:FH], 0.0)
    h1 = jnp.maximum(dinv * (a_ref[1] + hs_ref[1]) + b_ref[FH:H], 0.0)
    h = jnp.concatenate([h0, h1], axis=1)
    a_out_ref[...] = jnp.dot(h, wm1_ref[0:H, :],
                             preferred_element_type=jnp.float32) + bm1_ref[...]
    b_out_ref[...] = jnp.dot(h, wm1_ref[H:2 * H, :],
                             preferred_element_type=jnp.float32)


def _rows_spec():
    return pl.BlockSpec((_ROWS, H), lambda i: (i, 0))


def _split_spec():
    return pl.BlockSpec((2, _ROWS, FH), lambda i: (0, i, 0))


def _vec_spec():
    return pl.BlockSpec((_ROWS,), lambda i: (i,))


def _full_spec(shape):
    return pl.BlockSpec(shape, lambda i: tuple(0 for _ in shape))


def _tc1(deg0, deg1, x_p, W1):
    return pl.pallas_call(
        _tc1_body,
        grid=(_GRID,),
        in_specs=[_vec_spec(), _vec_spec(), _rows_spec(), _full_spec((D, H))],
        out_specs=[_vec_spec(), _split_spec()],
        out_shape=[jax.ShapeDtypeStruct((PN,), jnp.float32),
                   jax.ShapeDtypeStruct((2, PN, FH), jnp.float32)],
    )(deg0, deg1, x_p, W1)


def _tc2(agg, hs, dinv, b, W):
    return pl.pallas_call(
        _tc2_body,
        grid=(_GRID,),
        in_specs=[_split_spec(), _split_spec(), _vec_spec(),
                  _full_spec((H,)), _full_spec((H, H))],
        out_specs=_split_spec(),
        out_shape=jax.ShapeDtypeStruct((2, PN, FH), jnp.float32),
    )(agg, hs, dinv, b, W)


def _tc3(agg, hs, dinv, b, Wm1, bm1):
    return pl.pallas_call(
        _tc3_body,
        grid=(_GRID,),
        in_specs=[_split_spec(), _split_spec(), _vec_spec(),
                  _full_spec((H,)), _full_spec((2 * H, H)), _full_spec((H,))],
        out_specs=[_rows_spec(), _rows_spec()],
        out_shape=[jax.ShapeDtypeStruct((PN, H), jnp.float32),
                   jax.ShapeDtypeStruct((PN, H), jnp.float32)],
    )(agg, hs, dinv, b, Wm1, bm1)


def kernel(x, edge_index, W1, b1, W2, b2, Wm1, bm1, Wm2, bm2):
    src = edge_index[0]
    dst = edge_index[1]
    pad_ids = (N + (jnp.arange(PE - E, dtype=jnp.int32) % (PN - N)))
    psrc = jnp.concatenate([src, pad_ids])
    pdst = jnp.concatenate([dst, pad_ids])
    psrc3 = psrc.reshape(NW, NCH, K)
    pdst3 = pdst.reshape(NW, NCH, K)
    x_p = jnp.pad(x, ((0, PN - N), (0, 0)))

    psrc16 = psrc.reshape(16, NCH2, K)
    pdst16 = pdst.reshape(16, NCH2, K)

    deg_p = _sc_deg(pdst)
    dinv, hs1 = _tc1(deg_p[0], deg_p[1], x_p, W1)
    agg1 = _sc_agg(hs1, psrc16, pdst16)
    hs2 = _tc2(agg1, hs1, dinv, b1, W2)
    agg2 = _sc_agg(hs2, psrc16, pdst16)
    A, B = _tc3(agg2, hs2, dinv, b2, Wm1, bm1)
    partials = _sc_edge(A, B, psrc3, pdst3, Wm2.reshape(H))
    pflows = _tc4(partials, bm2)
    return pflows[:E]


# speedup vs baseline: 12.6231x; 1.0015x over previous
import functools

import jax
import jax.numpy as jnp
from jax import lax
from jax.experimental import pallas as pl
from jax.experimental.pallas import tpu as pltpu
from jax.experimental.pallas import tpu_sc as plsc

N = 10000
E = 320000
D = 128
H = 128

PN = 10240
NW = 32
EPW = 10240
PE = NW * EPW
K = 128
NCH = EPW // K
RPS = PN // 16
FH = H // 2
EPT = PE // 16
NCH2 = EPT // K

_mesh = plsc.VectorSubcoreMesh(core_axis_name="c", subcore_axis_name="s")


def _zero_2d(ref, rows, cols):
    z = jnp.zeros((16,), jnp.float32)
    g = cols // 16

    def body(i, _):
        r = i // g
        c = (i % g) * 16
        ref[r, pl.ds(c, 16)] = z
        return 0

    lax.fori_loop(0, rows * g, body, 0)


def _zero_1d(ref, n):
    z = jnp.zeros((16,), jnp.float32)

    def body(i, _):
        ref[pl.ds(i * 16, 16)] = z
        return 0

    lax.fori_loop(0, n // 16, body, 0)


@functools.partial(
    pl.kernel,
    out_type=jax.ShapeDtypeStruct((2, PN), jnp.float32),
    mesh=_mesh,
    compiler_params=pltpu.CompilerParams(needs_layout_passes=False),
    scratch_types=[
        pltpu.VMEM((K,), jnp.int32),
        pltpu.VMEM((K,), jnp.float32),
        pltpu.VMEM((RPS,), jnp.float32),
        pltpu.VMEM_SHARED((PN,), jnp.float32),
    ],
)
def _sc_deg(dst_hbm, out_hbm, idx_v, ones_v, stripe_v, deg_s):
    cid = lax.axis_index("c")
    sid = lax.axis_index("s")
    wid = cid * 16 + sid

    _zero_1d(stripe_v, RPS)
    pltpu.sync_copy(stripe_v, deg_s.at[pl.ds(sid * RPS, RPS)])
    o = jnp.ones((16,), jnp.float32)
    for i in range(K // 16):
        ones_v[pl.ds(i * 16, 16)] = o
    plsc.subcore_barrier()

    def chunk(c, _):
        base = wid * EPW + c * K
        pltpu.sync_copy(dst_hbm.at[pl.ds(base, K)], idx_v)
        pltpu.sync_copy(ones_v, deg_s.at[idx_v], add=True)
        return 0

    lax.fori_loop(0, NCH, chunk, 0)
    plsc.subcore_barrier()
    pltpu.sync_copy(deg_s.at[pl.ds(sid * RPS, RPS)], stripe_v)
    pltpu.sync_copy(stripe_v, out_hbm.at[cid, pl.ds(sid * RPS, RPS)])


@functools.partial(
    pl.kernel,
    out_type=jax.ShapeDtypeStruct((2, PN, FH), jnp.float32),
    mesh=_mesh,
    compiler_params=pltpu.CompilerParams(needs_layout_passes=False,
                                         use_tc_tiling_on_sc=False),
    scratch_types=[
        pltpu.VMEM((NCH2, K), jnp.int32),
        pltpu.VMEM((NCH2, K), jnp.int32),
        pltpu.VMEM((4, K, FH), jnp.float32),
        pltpu.VMEM_SHARED((PN, FH), jnp.float32),
        pltpu.SemaphoreType.DMA,
        pltpu.SemaphoreType.DMA,
        pltpu.SemaphoreType.DMA,
        pltpu.SemaphoreType.DMA,
        pltpu.SemaphoreType.DMA,
        pltpu.SemaphoreType.DMA,
        pltpu.SemaphoreType.DMA,
        pltpu.SemaphoreType.DMA,
    ],
)
def _sc_agg(table_hbm, src_hbm, dst_hbm, out_hbm, sidx_v, didx_v, rows_v,
            agg_s, g0, g1, g2, g3, s0, s1, s2, s3):
    cid = lax.axis_index("c")
    sid = lax.axis_index("s")
    gsem = [g0, g1, g2, g3]
    ssem = [s0, s1, s2, s3]
    table_c = table_hbm.at[cid]

    _zero_2d(rows_v.at[0], K, FH)
    for j in range(RPS // K):
        pltpu.sync_copy(rows_v.at[0], agg_s.at[pl.ds(sid * RPS + j * K, K)])
    pltpu.sync_copy(src_hbm.at[sid], sidx_v)
    pltpu.sync_copy(dst_hbm.at[sid], didx_v)
    plsc.subcore_barrier()

    def gather(c, b):
        pltpu.async_copy(table_c.at[sidx_v.at[c]], rows_v.at[b], gsem[b])

    def gwait(b):
        pltpu.make_async_copy(table_c.at[pl.ds(0, K)], rows_v.at[b],
                              gsem[b]).wait()

    def swait(b):
        pltpu.make_async_copy(rows_v.at[b], agg_s.at[pl.ds(0, K)],
                              ssem[b]).wait()

    def wave(w, sp, prefetch, first=False):
        b0, b1 = sp * 2, sp * 2 + 1
        o0, o1 = 2 - sp * 2, 3 - sp * 2
        gwait(b0)
        gwait(b1)
        if prefetch:
            if not first:
                swait(o0)
                swait(o1)
            gather(2 * w + 2, o0)
            gather(2 * w + 3, o1)
        pltpu.async_copy(rows_v.at[b0], agg_s.at[didx_v.at[2 * w]], ssem[b0],
                         add=True)
        pltpu.async_copy(rows_v.at[b1], agg_s.at[didx_v.at[2 * w + 1]],
                         ssem[b1], add=True)

    gather(0, 0)
    gather(1, 1)
    wave(0, 0, True, first=True)

    def pair(p, _):
        wave(2 * p + 1, 1, True)
        wave(2 * p + 2, 0, True)
        return 0

    lax.fori_loop(0, (NCH2 // 2 - 2) // 2, pair, 0)
    wave(NCH2 // 2 - 1, 1, False)
    for b in range(4):
        swait(b)

    plsc.subcore_barrier()
    for j in range(RPS // K):
        off = sid * RPS + j * K
        pltpu.sync_copy(agg_s.at[pl.ds(off, K)], rows_v.at[0])
        pltpu.sync_copy(rows_v.at[0], out_hbm.at[cid, pl.ds(off, K)])


@functools.partial(
    pl.kernel,
    out_type=jax.ShapeDtypeStruct((PE, 16), jnp.float32),
    mesh=_mesh,
    compiler_params=pltpu.CompilerParams(needs_layout_passes=False),
    scratch_types=[
        pltpu.VMEM((NCH, K), jnp.int32),
        pltpu.VMEM((NCH, K), jnp.int32),
        pltpu.VMEM((2, K, H), jnp.float32),
        pltpu.VMEM((2, K, H), jnp.float32),
        pltpu.VMEM((H,), jnp.float32),
        pltpu.VMEM((2, K, 16), jnp.float32),
        pltpu.SemaphoreType.DMA,
        pltpu.SemaphoreType.DMA,
        pltpu.SemaphoreType.DMA,
        pltpu.SemaphoreType.DMA,
        pltpu.SemaphoreType.DMA,
        pltpu.SemaphoreType.DMA,
    ],
)
def _sc_edge(a_hbm, b_hbm, src_hbm, dst_hbm, w2_hbm, out_hbm,
             ridx_v, cidx_v, ra_v, rb_v, w_v, o_v,
             ga0, ga1, gb0, gb1, os0, os1):
    cid = lax.axis_index("c")
    sid = lax.axis_index("s")
    wid = cid * 16 + sid
    gsa = [ga0, ga1]
    gsb = [gb0, gb1]
    osem = [os0, os1]

    pltpu.sync_copy(w2_hbm, w_v)
    pltpu.sync_copy(src_hbm.at[wid], ridx_v)
    pltpu.sync_copy(dst_hbm.at[wid], cidx_v)
    wvs = [w_v[pl.ds(jb * 16, 16)] for jb in range(H // 16)]

    def gather(c, s):
        pltpu.async_copy(a_hbm.at[ridx_v.at[c]], ra_v.at[s], gsa[s])
        pltpu.async_copy(b_hbm.at[cidx_v.at[c]], rb_v.at[s], gsb[s])

    def gwait(s):
        pltpu.make_async_copy(a_hbm.at[pl.ds(0, K)], ra_v.at[s], gsa[s]).wait()
        pltpu.make_async_copy(b_hbm.at[pl.ds(0, K)], rb_v.at[s], gsb[s]).wait()

    def owait(s):
        pltpu.make_async_copy(o_v.at[s], out_hbm.at[pl.ds(0, K)],
                              osem[s]).wait()

    def compute(s):
        ra = ra_v.at[s]
        rb = rb_v.at[s]
        ov = o_v.at[s]

        def pair(p, _):
            for u in range(2):
                e = p * 2 + u
                accs = [jnp.zeros((16,), jnp.float32) for _ in range(4)]
                for jb in range(H // 16):
                    a = ra[e, pl.ds(jb * 16, 16)]
                    b = rb[e, pl.ds(jb * 16, 16)]
                    t = jnp.maximum(a + b, 0.0)
                    accs[jb % 4] = accs[jb % 4] + t * wvs[jb]
                ov[e, :] = (accs[0] + accs[1]) + (accs[2] + accs[3])
            return 0

        lax.fori_loop(0, K // 2, pair, 0)

    def chunk(c, sp, prefetch, wait_out):
        gwait(sp)
        if prefetch:
            gather(c + 1, 1 - sp)
        if wait_out:
            owait(sp)
        compute(sp)
        pltpu.async_copy(o_v.at[sp], out_hbm.at[pl.ds(wid * EPW + c * K, K)],
                         osem[sp])

    gather(0, 0)
    chunk(0, 0, True, False)
    chunk(1, 1, True, False)

    def pair_of_chunks(p, _):
        chunk(2 * p + 2, 0, True, True)
        chunk(2 * p + 3, 1, True, True)
        return 0

    lax.fori_loop(0, (NCH - 4) // 2, pair_of_chunks, 0)
    chunk(NCH - 2, 0, True, True)
    chunk(NCH - 1, 1, False, True)
    owait(0)
    owait(1)


def _tc4_body(p_ref, bm2_ref, out_ref):
    out_ref[...] = jnp.sum(p_ref[...], axis=1, keepdims=True) + bm2_ref[0]


_EROWS = 4096


def _tc4(partials, bm2):
    return pl.pallas_call(
        _tc4_body,
        grid=(PE // _EROWS,),
        in_specs=[pl.BlockSpec((_EROWS, 16), lambda i: (i, 0)),
                  pl.BlockSpec((1,), lambda i: (0,))],
        out_specs=pl.BlockSpec((_EROWS, 1), lambda i: (i, 0)),
        out_shape=jax.ShapeDtypeStruct((PE, 1), jnp.float32),
    )(partials, bm2)


_ROWS = 1024
_GRID = PN // _ROWS


def _tc1a_body(x_ref, w1_ref, h_ref):
    h_ref[...] = jnp.dot(x_ref[...], w1_ref[...],
                         preferred_element_type=jnp.float32)


def _tc1_body(deg0_ref, deg1_ref, h_ref, dinv_ref, hs1_ref):
    deg = deg0_ref[...] + deg1_ref[...] + 1.0
    dinv = lax.rsqrt(deg)
    dinv_ref[...] = dinv
    h = h_ref[...] * dinv[:, None]
    hs1_ref[0] = h[:, 0:FH]
    hs1_ref[1] = h[:, FH:H]


def _tc2_body(a_ref, hs_ref, dinv_ref, b_ref, w_ref, out_ref):
    dinv = dinv_ref[...][:, None]
    h0 = jnp.maximum(dinv * (a_ref[0] + hs_ref[0]) + b_ref[0:FH], 0.0)
    h1 = jnp.maximum(dinv * (a_ref[1] + hs_ref[1]) + b_ref[FH:H], 0.0)
    h = jnp.concatenate([h0, h1], axis=1)
    t = jnp.dot(h, w_ref[...], preferred_element_type=jnp.float32) * dinv
    out_ref[0] = t[:, 0:FH]
    out_ref[1] = t[:, FH:H]


def _tc3_body(a_ref, hs_ref, dinv_ref, b_ref, wm1_ref, bm1_ref,
              a_out_ref, b_out_ref):
    dinv = dinv_ref[...][:, None]
    h0 = jnp.maximum(dinv * (a_ref[0] + hs_ref[0]) + b_ref[0:FH], 0.0)
    h1 = jnp.maximum(dinv * (a_ref[1] + hs_ref[1]) + b_ref[FH:H], 0.0)
    h = jnp.concatenate([h0, h1], axis=1)
    a_out_ref[...] = jnp.dot(h, wm1_ref[0:H, :],
                             preferred_element_type=jnp.float32) + bm1_ref[...]
    b_out_ref[...] = jnp.dot(h, wm1_ref[H:2 * H, :],
                             preferred_element_type=jnp.float32)


def _rows_spec():
    return pl.BlockSpec((_ROWS, H), lambda i: (i, 0))


def _split_spec():
    return pl.BlockSpec((2, _ROWS, FH), lambda i: (0, i, 0))


def _vec_spec():
    return pl.BlockSpec((_ROWS,), lambda i: (i,))


def _full_spec(shape):
    return pl.BlockSpec(shape, lambda i: tuple(0 for _ in shape))


def _tc1a(x_p, W1):
    return pl.pallas_call(
        _tc1a_body,
        grid=(_GRID,),
        in_specs=[_rows_spec(), _full_spec((D, H))],
        out_specs=_rows_spec(),
        out_shape=jax.ShapeDtypeStruct((PN, H), jnp.float32),
    )(x_p, W1)


def _tc1(deg0, deg1, h1raw, W1):
    return pl.pallas_call(
        _tc1_body,
        grid=(_GRID,),
        in_specs=[_vec_spec(), _vec_spec(), _rows_spec()],
        out_specs=[_vec_spec(), _split_spec()],
        out_shape=[jax.ShapeDtypeStruct((PN,), jnp.float32),
                   jax.ShapeDtypeStruct((2, PN, FH), jnp.float32)],
    )(deg0, deg1, h1raw)


def _tc2(agg, hs, dinv, b, W):
    return pl.pallas_call(
        _tc2_body,
        grid=(_GRID,),
        in_specs=[_split_spec(), _split_spec(), _vec_spec(),
                  _full_spec((H,)), _full_spec((H, H))],
        out_specs=_split_spec(),
        out_shape=jax.ShapeDtypeStruct((2, PN, FH), jnp.float32),
    )(agg, hs, dinv, b, W)


def _tc3(agg, hs, dinv, b, Wm1, bm1):
    return pl.pallas_call(
        _tc3_body,
        grid=(_GRID,),
        in_specs=[_split_spec(), _split_spec(), _vec_spec(),
                  _full_spec((H,)), _full_spec((2 * H, H)), _full_spec((H,))],
        out_specs=[_rows_spec(), _rows_spec()],
        out_shape=[jax.ShapeDtypeStruct((PN, H), jnp.float32),
                   jax.ShapeDtypeStruct((PN, H), jnp.float32)],
    )(agg, hs, dinv, b, Wm1, bm1)


def kernel(x, edge_index, W1, b1, W2, b2, Wm1, bm1, Wm2, bm2):
    src = edge_index[0]
    dst = edge_index[1]
    pad_ids = (N + (jnp.arange(PE - E, dtype=jnp.int32) % (PN - N)))
    psrc = jnp.concatenate([src, pad_ids])
    pdst = jnp.concatenate([dst, pad_ids])
    psrc3 = psrc.reshape(NW, NCH, K)
    pdst3 = pdst.reshape(NW, NCH, K)
    x_p = jnp.pad(x, ((0, PN - N), (0, 0)))

    psrc16 = psrc.reshape(16, NCH2, K)
    pdst16 = pdst.reshape(16, NCH2, K)

    h1raw = _tc1a(x_p, W1)
    deg_p = _sc_deg(pdst)
    dinv, hs1 = _tc1(deg_p[0], deg_p[1], h1raw, W1)
    agg1 = _sc_agg(hs1, psrc16, pdst16)
    hs2 = _tc2(agg1, hs1, dinv, b1, W2)
    agg2 = _sc_agg(hs2, psrc16, pdst16)
    A, B = _tc3(agg2, hs2, dinv, b2, Wm1, bm1)
    partials = _sc_edge(A, B, psrc3, pdst3, Wm2.reshape(H))
    pflows = _tc4(partials, bm2)
    return pflows[:E]


# speedup vs baseline: 14.0730x; 1.1149x over previous
import functools

import jax
import jax.numpy as jnp
from jax import lax
from jax.experimental import pallas as pl
from jax.experimental.pallas import tpu as pltpu
from jax.experimental.pallas import tpu_sc as plsc

N = 10000
E = 320000
D = 128
H = 128

PN = 10240
NW = 32
EPW = 10240
PE = NW * EPW
K = 128
NCH = EPW // K
RPS = PN // 16
FH = H // 2
EPT = PE // 16
NCH2 = EPT // K

_mesh = plsc.VectorSubcoreMesh(core_axis_name="c", subcore_axis_name="s")


def _zero_2d(ref, rows, cols):
    z = jnp.zeros((16,), jnp.float32)
    g = cols // 16

    def body(i, _):
        r = i // g
        c = (i % g) * 16
        ref[r, pl.ds(c, 16)] = z
        return 0

    lax.fori_loop(0, rows * g, body, 0)


def _zero_1d(ref, n):
    z = jnp.zeros((16,), jnp.float32)

    def body(i, _):
        ref[pl.ds(i * 16, 16)] = z
        return 0

    lax.fori_loop(0, n // 16, body, 0)


@functools.partial(
    pl.kernel,
    out_type=jax.ShapeDtypeStruct((2, PN), jnp.float32),
    mesh=_mesh,
    compiler_params=pltpu.CompilerParams(needs_layout_passes=False),
    scratch_types=[
        pltpu.VMEM((K,), jnp.int32),
        pltpu.VMEM((K,), jnp.float32),
        pltpu.VMEM((RPS,), jnp.float32),
        pltpu.VMEM_SHARED((PN,), jnp.float32),
    ],
)
def _sc_deg(dst_hbm, out_hbm, idx_v, ones_v, stripe_v, deg_s):
    cid = lax.axis_index("c")
    sid = lax.axis_index("s")
    wid = cid * 16 + sid

    _zero_1d(stripe_v, RPS)
    pltpu.sync_copy(stripe_v, deg_s.at[pl.ds(sid * RPS, RPS)])
    o = jnp.ones((16,), jnp.float32)
    for i in range(K // 16):
        ones_v[pl.ds(i * 16, 16)] = o
    plsc.subcore_barrier()

    def chunk(c, _):
        base = wid * EPW + c * K
        pltpu.sync_copy(dst_hbm.at[pl.ds(base, K)], idx_v)
        pltpu.sync_copy(ones_v, deg_s.at[idx_v], add=True)
        return 0

    lax.fori_loop(0, NCH, chunk, 0)
    plsc.subcore_barrier()
    pltpu.sync_copy(deg_s.at[pl.ds(sid * RPS, RPS)], stripe_v)
    pltpu.sync_copy(stripe_v, out_hbm.at[cid, pl.ds(sid * RPS, RPS)])


@functools.partial(
    pl.kernel,
    out_type=jax.ShapeDtypeStruct((2, PN, FH), jnp.float32),
    mesh=_mesh,
    compiler_params=pltpu.CompilerParams(needs_layout_passes=False,
                                         use_tc_tiling_on_sc=False),
    scratch_types=[
        pltpu.VMEM((NCH2, K), jnp.int32),
        pltpu.VMEM((NCH2, K), jnp.int32),
        pltpu.VMEM((4, K, FH), jnp.float32),
        pltpu.VMEM_SHARED((PN, FH), jnp.float32),
        pltpu.SemaphoreType.DMA,
        pltpu.SemaphoreType.DMA,
        pltpu.SemaphoreType.DMA,
        pltpu.SemaphoreType.DMA,
        pltpu.SemaphoreType.DMA,
        pltpu.SemaphoreType.DMA,
        pltpu.SemaphoreType.DMA,
        pltpu.SemaphoreType.DMA,
    ],
)
def _sc_agg(table_hbm, src_hbm, dst_hbm, out_hbm, sidx_v, didx_v, rows_v,
            agg_s, g0, g1, g2, g3, s0, s1, s2, s3):
    cid = lax.axis_index("c")
    sid = lax.axis_index("s")
    gsem = [g0, g1, g2, g3]
    ssem = [s0, s1, s2, s3]
    table_c = table_hbm.at[cid]

    _zero_2d(rows_v.at[0], K, FH)
    for j in range(RPS // K):
        pltpu.sync_copy(rows_v.at[0], agg_s.at[pl.ds(sid * RPS + j * K, K)])
    pltpu.sync_copy(src_hbm.at[sid], sidx_v)
    pltpu.sync_copy(dst_hbm.at[sid], didx_v)
    plsc.subcore_barrier()

    def gather(c, b):
        pltpu.async_copy(table_c.at[sidx_v.at[c]], rows_v.at[b], gsem[b])

    def gwait(b):
        pltpu.make_async_copy(table_c.at[pl.ds(0, K)], rows_v.at[b],
                              gsem[b]).wait()

    def swait(b):
        pltpu.make_async_copy(rows_v.at[b], agg_s.at[pl.ds(0, K)],
                              ssem[b]).wait()

    def wave(w, sp, prefetch, first=False):
        b0, b1 = sp * 2, sp * 2 + 1
        o0, o1 = 2 - sp * 2, 3 - sp * 2
        gwait(b0)
        gwait(b1)
        if prefetch:
            if not first:
                swait(o0)
                swait(o1)
            gather(2 * w + 2, o0)
            gather(2 * w + 3, o1)
        pltpu.async_copy(rows_v.at[b0], agg_s.at[didx_v.at[2 * w]], ssem[b0],
                         add=True)
        pltpu.async_copy(rows_v.at[b1], agg_s.at[didx_v.at[2 * w + 1]],
                         ssem[b1], add=True)

    gather(0, 0)
    gather(1, 1)
    wave(0, 0, True, first=True)

    def pair(p, _):
        wave(2 * p + 1, 1, True)
        wave(2 * p + 2, 0, True)
        return 0

    lax.fori_loop(0, (NCH2 // 2 - 2) // 2, pair, 0)
    wave(NCH2 // 2 - 1, 1, False)
    for b in range(4):
        swait(b)

    plsc.subcore_barrier()
    for j in range(RPS // K):
        off = sid * RPS + j * K
        pltpu.sync_copy(agg_s.at[pl.ds(off, K)], rows_v.at[0])
        pltpu.sync_copy(rows_v.at[0], out_hbm.at[cid, pl.ds(off, K)])


@functools.partial(
    pl.kernel,
    out_type=jax.ShapeDtypeStruct((PE, 16), jnp.float32),
    mesh=_mesh,
    compiler_params=pltpu.CompilerParams(needs_layout_passes=False),
    scratch_types=[
        pltpu.VMEM((NCH, K), jnp.int32),
        pltpu.VMEM((NCH, K), jnp.int32),
        pltpu.VMEM((2, K, H), jnp.float32),
        pltpu.VMEM((2, K, H), jnp.float32),
        pltpu.VMEM((H,), jnp.float32),
        pltpu.VMEM((2, K, 16), jnp.float32),
        pltpu.SemaphoreType.DMA,
        pltpu.SemaphoreType.DMA,
        pltpu.SemaphoreType.DMA,
        pltpu.SemaphoreType.DMA,
        pltpu.SemaphoreType.DMA,
        pltpu.SemaphoreType.DMA,
    ],
)
def _sc_edge(a_hbm, b_hbm, src_hbm, dst_hbm, w2_hbm, out_hbm,
             ridx_v, cidx_v, ra_v, rb_v, w_v, o_v,
             ga0, ga1, gb0, gb1, os0, os1):
    cid = lax.axis_index("c")
    sid = lax.axis_index("s")
    wid = cid * 16 + sid
    gsa = [ga0, ga1]
    gsb = [gb0, gb1]
    osem = [os0, os1]

    pltpu.sync_copy(w2_hbm, w_v)
    pltpu.sync_copy(src_hbm.at[wid], ridx_v)
    pltpu.sync_copy(dst_hbm.at[wid], cidx_v)
    wvs = [w_v[pl.ds(jb * 16, 16)] for jb in range(H // 16)]

    def gather(c, s):
        pltpu.async_copy(a_hbm.at[ridx_v.at[c]], ra_v.at[s], gsa[s])
        pltpu.async_copy(b_hbm.at[cidx_v.at[c]], rb_v.at[s], gsb[s])

    def gwait(s):
        pltpu.make_async_copy(a_hbm.at[pl.ds(0, K)], ra_v.at[s], gsa[s]).wait()
        pltpu.make_async_copy(b_hbm.at[pl.ds(0, K)], rb_v.at[s], gsb[s]).wait()

    def owait(s):
        pltpu.make_async_copy(o_v.at[s], out_hbm.at[pl.ds(0, K)],
                              osem[s]).wait()

    def compute(s):
        ra = ra_v.at[s]
        rb = rb_v.at[s]
        ov = o_v.at[s]

        def pair(p, _):
            for u in range(2):
                e = p * 2 + u
                accs = [jnp.zeros((16,), jnp.float32) for _ in range(4)]
                for jb in range(H // 16):
                    a = ra[e, pl.ds(jb * 16, 16)]
                    b = rb[e, pl.ds(jb * 16, 16)]
                    t = jnp.maximum(a + b, 0.0)
                    accs[jb % 4] = accs[jb % 4] + t * wvs[jb]
                ov[e, :] = (accs[0] + accs[1]) + (accs[2] + accs[3])
            return 0

        lax.fori_loop(0, K // 2, pair, 0)

    def chunk(c, sp, prefetch, wait_out):
        gwait(sp)
        if prefetch:
            gather(c + 1, 1 - sp)
        if wait_out:
            owait(sp)
        compute(sp)
        pltpu.async_copy(o_v.at[sp], out_hbm.at[pl.ds(wid * EPW + c * K, K)],
                         osem[sp])

    gather(0, 0)
    chunk(0, 0, True, False)
    chunk(1, 1, True, False)

    def pair_of_chunks(p, _):
        chunk(2 * p + 2, 0, True, True)
        chunk(2 * p + 3, 1, True, True)
        return 0

    lax.fori_loop(0, (NCH - 4) // 2, pair_of_chunks, 0)
    chunk(NCH - 2, 0, True, True)
    chunk(NCH - 1, 1, False, True)
    owait(0)
    owait(1)


def _tc4_body(p_ref, bm2_ref, out_ref):
    out_ref[...] = jnp.sum(p_ref[...], axis=1, keepdims=True) + bm2_ref[0]


_EROWS = 3200


def _tc4(partials, bm2):
    return pl.pallas_call(
        _tc4_body,
        grid=(E // _EROWS,),
        in_specs=[pl.BlockSpec((_EROWS, 16), lambda i: (i, 0)),
                  pl.BlockSpec((1,), lambda i: (0,))],
        out_specs=pl.BlockSpec((_EROWS, 1), lambda i: (i, 0)),
        out_shape=jax.ShapeDtypeStruct((E, 1), jnp.float32),
    )(partials, bm2)


_ROWS = 1024
_GRID = PN // _ROWS


def _tc1a_body(x_ref, w1_ref, h_ref):
    h_ref[...] = jnp.dot(x_ref[...], w1_ref[...],
                         preferred_element_type=jnp.float32)


def _tc1_body(deg0_ref, deg1_ref, h_ref, dinv_ref, hs1_ref):
    deg = deg0_ref[...] + deg1_ref[...] + 1.0
    dinv = lax.rsqrt(deg)
    dinv_ref[...] = dinv
    h = h_ref[...] * dinv[:, None]
    hs1_ref[0] = h[:, 0:FH]
    hs1_ref[1] = h[:, FH:H]


def _tc2_body(a_ref, hs_ref, dinv_ref, b_ref, w_ref, out_ref):
    dinv = dinv_ref[...][:, None]
    h0 = jnp.maximum(dinv * (a_ref[0] + hs_ref[0]) + b_ref[0:FH], 0.0)
    h1 = jnp.maximum(dinv * (a_ref[1] + hs_ref[1]) + b_ref[FH:H], 0.0)
    h = jnp.concatenate([h0, h1], axis=1)
    t = jnp.dot(h, w_ref[...], preferred_element_type=jnp.float32) * dinv
    out_ref[0] = t[:, 0:FH]
    out_ref[1] = t[:, FH:H]


def _tc3_body(a_ref, hs_ref, dinv_ref, b_ref, wm1_ref, bm1_ref,
              a_out_ref, b_out_ref):
    dinv = dinv_ref[...][:, None]
    h0 = jnp.maximum(dinv * (a_ref[0] + hs_ref[0]) + b_ref[0:FH], 0.0)
    h1 = jnp.maximum(dinv * (a_ref[1] + hs_ref[1]) + b_ref[FH:H], 0.0)
    h = jnp.concatenate([h0, h1], axis=1)
    a_out_ref[...] = jnp.dot(h, wm1_ref[0:H, :],
                             preferred_element_type=jnp.float32) + bm1_ref[...]
    b_out_ref[...] = jnp.dot(h, wm1_ref[H:2 * H, :],
                             preferred_element_type=jnp.float32)


def _rows_spec():
    return pl.BlockSpec((_ROWS, H), lambda i: (i, 0))


def _split_spec():
    return pl.BlockSpec((2, _ROWS, FH), lambda i: (0, i, 0))


def _vec_spec():
    return pl.BlockSpec((_ROWS,), lambda i: (i,))


def _full_spec(shape):
    return pl.BlockSpec(shape, lambda i: tuple(0 for _ in shape))


def _tc1a(x_p, W1):
    return pl.pallas_call(
        _tc1a_body,
        grid=(_GRID,),
        in_specs=[_rows_spec(), _full_spec((D, H))],
        out_specs=_rows_spec(),
        out_shape=jax.ShapeDtypeStruct((PN, H), jnp.float32),
    )(x_p, W1)


def _tc1(deg0, deg1, h1raw, W1):
    return pl.pallas_call(
        _tc1_body,
        grid=(_GRID,),
        in_specs=[_vec_spec(), _vec_spec(), _rows_spec()],
        out_specs=[_vec_spec(), _split_spec()],
        out_shape=[jax.ShapeDtypeStruct((PN,), jnp.float32),
                   jax.ShapeDtypeStruct((2, PN, FH), jnp.float32)],
    )(deg0, deg1, h1raw)


def _tc2(agg, hs, dinv, b, W):
    return pl.pallas_call(
        _tc2_body,
        grid=(_GRID,),
        in_specs=[_split_spec(), _split_spec(), _vec_spec(),
                  _full_spec((H,)), _full_spec((H, H))],
        out_specs=_split_spec(),
        out_shape=jax.ShapeDtypeStruct((2, PN, FH), jnp.float32),
    )(agg, hs, dinv, b, W)


def _tc3(agg, hs, dinv, b, Wm1, bm1):
    return pl.pallas_call(
        _tc3_body,
        grid=(_GRID,),
        in_specs=[_split_spec(), _split_spec(), _vec_spec(),
                  _full_spec((H,)), _full_spec((2 * H, H)), _full_spec((H,))],
        out_specs=[_rows_spec(), _rows_spec()],
        out_shape=[jax.ShapeDtypeStruct((PN, H), jnp.float32),
                   jax.ShapeDtypeStruct((PN, H), jnp.float32)],
    )(agg, hs, dinv, b, Wm1, bm1)


def kernel(x, edge_index, W1, b1, W2, b2, Wm1, bm1, Wm2, bm2):
    src = edge_index[0]
    dst = edge_index[1]
    pad_ids = (N + (jnp.arange(PE - E, dtype=jnp.int32) % (PN - N)))
    psrc = jnp.concatenate([src, pad_ids])
    pdst = jnp.concatenate([dst, pad_ids])
    psrc3 = psrc.reshape(NW, NCH, K)
    pdst3 = pdst.reshape(NW, NCH, K)
    x_p = jnp.pad(x, ((0, PN - N), (0, 0)))

    psrc16 = psrc.reshape(16, NCH2, K)
    pdst16 = pdst.reshape(16, NCH2, K)

    h1raw = _tc1a(x_p, W1)
    deg_p = _sc_deg(pdst)
    dinv, hs1 = _tc1(deg_p[0], deg_p[1], h1raw, W1)
    agg1 = _sc_agg(hs1, psrc16, pdst16)
    hs2 = _tc2(agg1, hs1, dinv, b1, W2)
    agg2 = _sc_agg(hs2, psrc16, pdst16)
    A, B = _tc3(agg2, hs2, dinv, b2, Wm1, bm1)
    partials = _sc_edge(A, B, psrc3, pdst3, Wm2.reshape(H))
    return _tc4(partials, bm2)
